# baseline (speedup 1.0000x reference)
import functools

import jax
import jax.numpy as jnp
from jax import lax
from jax.experimental import pallas as pl
from jax.experimental.pallas import tpu as pltpu
from jax.experimental.pallas import tpu_sc as plsc

HIDDEN_DIM = 512
BATCH = 16384
LANES = 16

_info = plsc.get_sparse_core_info()
NC, NS = _info.num_cores, _info.num_subcores
NW = NC * NS
B_PER_W = BATCH // NW
CHUNK = 32
N_CHUNKS = B_PER_W // CHUNK
NCOL = 4
CW = HIDDEN_DIM // NCOL
JH = CW // LANES
GRPS = CHUNK // LANES


def _perm(x, idx):
    return lax.gather(
        x, idx.reshape(LANES, 1),
        lax.GatherDimensionNumbers(
            offset_dims=(), collapsed_slice_dims=(0,), start_index_map=(0,)),
        (1,), mode=lax.GatherScatterMode.PROMISE_IN_BOUNDS)


def _mesh_kernel():
    mesh = plsc.VectorSubcoreMesh(core_axis_name="c", subcore_axis_name="s")

    @functools.partial(
        pl.kernel,
        mesh=mesh,
        out_type=jax.ShapeDtypeStruct((BATCH, HIDDEN_DIM), jnp.float32),
        scratch_types=[
            pltpu.VMEM((B_PER_W,), jnp.int32),
            pltpu.VMEM((2, HIDDEN_DIM), jnp.float32),
            pltpu.VMEM((CHUNK, HIDDEN_DIM), jnp.float32),
            pltpu.VMEM((CHUNK, HIDDEN_DIM), jnp.float32),
            pltpu.SemaphoreType.DMA,
            pltpu.SemaphoreType.DMA,
        ],
    )
    def body(table_hbm, idx_hbm, out_hbm, idx_v, tab_v, rows0, rows1,
             sem0, sem1):
        wid = lax.axis_index("s") * NC + lax.axis_index("c")
        base = wid * B_PER_W
        pltpu.sync_copy(idx_hbm.at[wid], idx_v)
        pltpu.sync_copy(table_hbm, tab_v)

        lane0 = lax.iota(jnp.int32, LANES) * 0
        bufs = (rows0, rows1)
        sems = (sem0, sem1)

        def compute_chunk(k, buf):
            for h in range(NCOL):
                c0 = h * CW
                w0 = [tab_v[0, pl.ds(c0 + j * LANES, LANES)]
                      for j in range(JH)]
                dif = [tab_v[1, pl.ds(c0 + j * LANES, LANES)] - w0[j]
                       for j in range(JH)]
                for g in range(GRPS):
                    v = idx_v[pl.ds(k * CHUNK + g * LANES, LANES)]
                    vf = v.astype(jnp.float32)
                    for r in range(LANES):
                        f = _perm(vf, lane0 + r)
                        row = g * LANES + r
                        for j in range(JH):
                            buf[row, pl.ds(c0 + j * LANES, LANES)] = (
                                w0[j] + f * dif[j])

        def outer(kk, _):
            for b in range(2):
                k = kk * 2 + b

                @pl.when(kk > 0)
                def _(b=b):
                    pltpu.make_async_copy(
                        bufs[b], out_hbm.at[pl.ds(base, CHUNK)],
                        sems[b]).wait()

                compute_chunk(k, bufs[b])
                pltpu.async_copy(
                    bufs[b], out_hbm.at[pl.ds(base + k * CHUNK, CHUNK)],
                    sems[b])
            return 0

        lax.fori_loop(0, N_CHUNKS // 2, outer, 0)
        for b in range(2):
            pltpu.make_async_copy(
                bufs[b], out_hbm.at[pl.ds(base, CHUNK)], sems[b]).wait()

    return body


_sc_lookup = _mesh_kernel()


@jax.jit
def kernel(domain_ids, embed_weight):
    ids = domain_ids.astype(jnp.int32).reshape(NW, B_PER_W)
    return _sc_lookup(embed_weight, ids)


# speedup vs baseline: 1.0898x; 1.0898x over previous
import functools

import jax
import jax.numpy as jnp
from jax import lax
from jax.experimental import pallas as pl
from jax.experimental.pallas import tpu as pltpu
from jax.experimental.pallas import tpu_sc as plsc

HIDDEN_DIM = 512
BATCH = 16384
LANES = 16

_info = plsc.get_sparse_core_info()
NC, NS = _info.num_cores, _info.num_subcores
NW = NC * NS
B_PER_W = BATCH // NW

CHUNK = 32
N_CHUNKS = 6
CH_ROWS = CHUNK * N_CHUNKS
DMA_ROWS = B_PER_W - CH_ROWS
DMA_GRPS = DMA_ROWS // LANES

NCOL = 4
CW = HIDDEN_DIM // NCOL
JH = CW // LANES
GRPS = CHUNK // LANES


def _perm(x, idx):
    return lax.gather(
        x, idx.reshape(LANES, 1),
        lax.GatherDimensionNumbers(
            offset_dims=(), collapsed_slice_dims=(0,), start_index_map=(0,)),
        (1,), mode=lax.GatherScatterMode.PROMISE_IN_BOUNDS)


def _mesh_kernel():
    mesh = plsc.VectorSubcoreMesh(core_axis_name="c", subcore_axis_name="s")

    @functools.partial(
        pl.kernel,
        mesh=mesh,
        out_type=jax.ShapeDtypeStruct((BATCH, HIDDEN_DIM), jnp.float32),
        scratch_types=[
            pltpu.VMEM((B_PER_W,), jnp.int32),
            pltpu.VMEM((2, HIDDEN_DIM), jnp.float32),
            pltpu.VMEM((CHUNK, HIDDEN_DIM), jnp.float32),
            pltpu.VMEM((CHUNK, HIDDEN_DIM), jnp.float32),
            pltpu.SemaphoreType.DMA,
            pltpu.SemaphoreType.DMA,
            pltpu.SemaphoreType.DMA,
        ],
    )
    def body(table_hbm, idx_hbm, out_hbm, idx_v, tab_v, rows0, rows1,
             sem0, sem1, semr):
        wid = lax.axis_index("s") * NC + lax.axis_index("c")
        base = wid * B_PER_W
        pltpu.sync_copy(idx_hbm.at[wid], idx_v)
        pltpu.sync_copy(table_hbm, tab_v)

        lane0 = lax.iota(jnp.int32, LANES) * 0
        bufs = (rows0, rows1)
        sems = (sem0, sem1)

        def row_grp(t, _):
            v = idx_v[pl.ds(CH_ROWS + t * LANES, LANES)]
            row0 = base + CH_ROWS + t * LANES
            for r in range(LANES):
                pltpu.async_copy(tab_v.at[v[r]], out_hbm.at[row0 + r], semr)
            return 0

        lax.fori_loop(0, DMA_GRPS, row_grp, 0)

        def compute_chunk(k, buf):
            for h in range(NCOL):
                c0 = h * CW
                w0 = [tab_v[0, pl.ds(c0 + j * LANES, LANES)]
                      for j in range(JH)]
                dif = [tab_v[1, pl.ds(c0 + j * LANES, LANES)] - w0[j]
                       for j in range(JH)]
                for g in range(GRPS):
                    v = idx_v[pl.ds(k * CHUNK + g * LANES, LANES)]
                    vf = v.astype(jnp.float32)
                    for r in range(LANES):
                        f = _perm(vf, lane0 + r)
                        row = g * LANES + r
                        for j in range(JH):
                            buf[row, pl.ds(c0 + j * LANES, LANES)] = (
                                w0[j] + f * dif[j])

        def outer(kk, _):
            for b in range(2):
                k = kk * 2 + b

                @pl.when(kk > 0)
                def _(b=b):
                    pltpu.make_async_copy(
                        bufs[b], out_hbm.at[pl.ds(base, CHUNK)],
                        sems[b]).wait()

                compute_chunk(k, bufs[b])
                pltpu.async_copy(
                    bufs[b], out_hbm.at[pl.ds(base + k * CHUNK, CHUNK)],
                    sems[b])
            return 0

        lax.fori_loop(0, N_CHUNKS // 2, outer, 0)
        for b in range(2):
            pltpu.make_async_copy(
                bufs[b], out_hbm.at[pl.ds(base, CHUNK)], sems[b]).wait()

        def drain_row(t, _):
            pltpu.make_async_copy(
                tab_v.at[0], out_hbm.at[base], semr).wait()
            return 0

        lax.fori_loop(0, DMA_ROWS, drain_row, 0)

    return body


_sc_lookup = _mesh_kernel()


@jax.jit
def kernel(domain_ids, embed_weight):
    ids = domain_ids.astype(jnp.int32).reshape(NW, B_PER_W)
    return _sc_lookup(embed_weight, ids)


# speedup vs baseline: 1.4203x; 1.3033x over previous
import functools

import jax
import jax.numpy as jnp
from jax import lax
from jax.experimental import pallas as pl
from jax.experimental.pallas import tpu as pltpu
from jax.experimental.pallas import tpu_sc as plsc

HIDDEN_DIM = 512
BATCH = 16384
LANES = 16

_info = plsc.get_sparse_core_info()
NC, NS = _info.num_cores, _info.num_subcores
NW = NC * NS
B_PER_W = BATCH // NW

QROWS = 4
NQPAT = 2 ** QROWS
NGRP = B_PER_W // LANES
QPG = LANES // QROWS
N_Q = B_PER_W // QROWS

NCOL = 4
CW = HIDDEN_DIM // NCOL
JH = CW // LANES


def _mesh_kernel():
    mesh = plsc.VectorSubcoreMesh(core_axis_name="c", subcore_axis_name="s")

    @functools.partial(
        pl.kernel,
        mesh=mesh,
        out_type=jax.ShapeDtypeStruct((BATCH, HIDDEN_DIM), jnp.float32),
        scratch_types=[
            pltpu.VMEM((B_PER_W,), jnp.int32),
            pltpu.VMEM((2, HIDDEN_DIM), jnp.float32),
            pltpu.VMEM((NQPAT, QROWS, HIDDEN_DIM), jnp.float32),
            pltpu.SemaphoreType.DMA,
        ],
    )
    def body(table_hbm, idx_hbm, out_hbm, idx_v, tab_v, quads, sem):
        wid = lax.axis_index("s") * NC + lax.axis_index("c")
        base = wid * B_PER_W
        pltpu.sync_copy(idx_hbm.at[wid], idx_v)
        pltpu.sync_copy(table_hbm, tab_v)

        for h in range(NCOL):
            c0 = h * CW
            w0 = [tab_v[0, pl.ds(c0 + j * LANES, LANES)] for j in range(JH)]
            w1 = [tab_v[1, pl.ds(c0 + j * LANES, LANES)] for j in range(JH)]
            for q in range(NQPAT):
                for rr in range(QROWS):
                    src = w1 if (q >> (QROWS - 1 - rr)) & 1 else w0
                    for j in range(JH):
                        quads[q, rr, pl.ds(c0 + j * LANES, LANES)] = src[j]

        def grp_body(t, _):
            v = idx_v[pl.ds(t * LANES, LANES)]
            row0 = base + t * LANES
            for i in range(QPG):
                pat = (v[4 * i] * 8 + v[4 * i + 1] * 4
                       + v[4 * i + 2] * 2 + v[4 * i + 3])
                pltpu.async_copy(
                    quads.at[pat],
                    out_hbm.at[pl.ds(row0 + i * QROWS, QROWS)], sem)
            return 0

        lax.fori_loop(0, NGRP, grp_body, 0)

        def drain_body(t, _):
            pltpu.make_async_copy(
                quads.at[0], out_hbm.at[pl.ds(base, QROWS)], sem).wait()
            return 0

        lax.fori_loop(0, N_Q, drain_body, 0)

    return body


_sc_lookup = _mesh_kernel()


@jax.jit
def kernel(domain_ids, embed_weight):
    ids = domain_ids.astype(jnp.int32).reshape(NW, B_PER_W)
    return _sc_lookup(embed_weight, ids)
